# baseline (speedup 1.0000x reference)
import functools

import jax
import jax.numpy as jnp
import numpy as np
from jax import lax
from jax.experimental import pallas as pl
from jax.experimental.pallas import tpu as pltpu
from jax.experimental.pallas import tpu_sc as plsc

_N_CAT = 26
_CAT_DIM = 32
_N_MER = 2
_MER_DIM = 32
_B = 4096
_HIST = 50
_N_NUM = 13
_H1, _H2, _OUT = 512, 256, 2
_FEAT = _N_CAT * _CAT_DIM + _N_MER * _MER_DIM

_NC, _NS = 2, 16
_NW = _NC * _NS
_BPW = _B // _NW
_CB = 8
_NCHUNK = _BPW // _CB
_PIECES = ((0, 128), (128, 128), (256, 128), (384, 16))

_mesh = plsc.VectorSubcoreMesh(
    core_axis_name="c", subcore_axis_name="s", num_cores=_NC, num_subcores=_NS
)


@functools.partial(
    pl.kernel,
    out_type=jax.ShapeDtypeStruct((_B, _FEAT), jnp.float32),
    mesh=_mesh,
    compiler_params=pltpu.CompilerParams(use_tc_tiling_on_sc=False),
    scratch_types=[
        pltpu.VMEM((_BPW,), jnp.int32),
        pltpu.VMEM((_BPW, _CAT_DIM), jnp.float32),
        pltpu.VMEM((_BPW, 128), jnp.float32),
        pltpu.VMEM((_CB * _HIST,), jnp.int32),
        pltpu.VMEM((_CB * _HIST, _MER_DIM), jnp.float32),
        pltpu.SemaphoreType.DMA,
    ],
)
def _sc_gather(*refs):
    xcat = refs[0:_N_CAT]
    xmer = refs[_N_CAT:_N_CAT + _N_MER]
    cat_tabs = refs[_N_CAT + _N_MER:2 * _N_CAT + _N_MER]
    mer_tabs = refs[2 * _N_CAT + _N_MER:2 * _N_CAT + 2 * _N_MER]
    out = refs[2 * _N_CAT + 2 * _N_MER]
    cidx_v, crow_v, grp_v, midx_v, mrow_v, sem = refs[2 * _N_CAT + 2 * _N_MER + 1:]

    wid = lax.axis_index("s") * _NC + lax.axis_index("c")
    base = wid * _BPW
    inv = np.float32(1.0 / _HIST)

    for g in range(_FEAT // 128):
        for f in range(4 * g, 4 * g + 4):
            slot = (f % 4) * 32
            if f < _N_CAT:
                pltpu.sync_copy(xcat[f].at[pl.ds(base, _BPW)], cidx_v)
                pltpu.async_copy(cat_tabs[f].at[cidx_v], crow_v, sem).wait()

                def _ccopy(r, carry, slot=slot):
                    grp_v[r, slot:slot + 16] = crow_v[r, 0:16]
                    grp_v[r, slot + 16:slot + 32] = crow_v[r, 16:32]
                    return carry

                lax.fori_loop(0, _BPW, _ccopy, 0)
            else:
                j = f - _N_CAT

                def _chunk(t, carry, j=j, slot=slot):
                    start = base * _HIST + t * (_CB * _HIST)
                    pltpu.sync_copy(xmer[j].at[pl.ds(start, _CB * _HIST)],
                                    midx_v)
                    copies = [
                        pltpu.async_copy(
                            mer_tabs[j].at[midx_v.at[pl.ds(off, ln)]],
                            mrow_v.at[pl.ds(off, ln)],
                            sem,
                        )
                        for off, ln in _PIECES
                    ]
                    for c in copies:
                        c.wait()

                    def _bag(b, carry2):
                        r0 = b * _HIST
                        a0 = mrow_v[r0, 0:16]
                        a1 = mrow_v[r0, 16:32]
                        b0 = mrow_v[r0 + 1, 0:16]
                        b1 = mrow_v[r0 + 1, 16:32]
                        for h in range(2, _HIST, 2):
                            a0 += mrow_v[r0 + h, 0:16]
                            a1 += mrow_v[r0 + h, 16:32]
                            b0 += mrow_v[r0 + h + 1, 0:16]
                            b1 += mrow_v[r0 + h + 1, 16:32]
                        row = t * _CB + b
                        grp_v[row, slot:slot + 16] = (a0 + b0) * inv
                        grp_v[row, slot + 16:slot + 32] = (a1 + b1) * inv
                        return carry2

                    lax.fori_loop(0, _CB, _bag, 0)
                    return carry

                lax.fori_loop(0, _NCHUNK, _chunk, 0)
        pltpu.sync_copy(grp_v, out.at[pl.ds(base, _BPW), pl.ds(g * 128, 128)])


_BT = 256


def _mlp_body(x_ref, xn_ref, w1a_ref, w1b_ref, b1_ref, w2_ref, b2_ref,
              w3_ref, b3_ref, o_ref):
    h = jnp.dot(x_ref[...], w1a_ref[...], preferred_element_type=jnp.float32)
    h += jnp.dot(xn_ref[...], w1b_ref[...], preferred_element_type=jnp.float32)
    h = jnp.maximum(h + b1_ref[...], 0.0)
    h = jnp.maximum(
        jnp.dot(h, w2_ref[...], preferred_element_type=jnp.float32) + b2_ref[...],
        0.0,
    )
    o_ref[...] = (
        jnp.dot(h, w3_ref[...], preferred_element_type=jnp.float32) + b3_ref[...]
    )


_mlp = pl.pallas_call(
    _mlp_body,
    grid=(_B // _BT,),
    in_specs=[
        pl.BlockSpec((_BT, _FEAT), lambda i: (i, 0)),
        pl.BlockSpec((_BT, _N_NUM), lambda i: (i, 0)),
        pl.BlockSpec((_FEAT, _H1), lambda i: (0, 0)),
        pl.BlockSpec((_N_NUM, _H1), lambda i: (0, 0)),
        pl.BlockSpec((_H1,), lambda i: (0,)),
        pl.BlockSpec((_H1, _H2), lambda i: (0, 0)),
        pl.BlockSpec((_H2,), lambda i: (0,)),
        pl.BlockSpec((_H2, _OUT), lambda i: (0, 0)),
        pl.BlockSpec((_OUT,), lambda i: (0,)),
    ],
    out_specs=pl.BlockSpec((_BT, _OUT), lambda i: (i, 0)),
    out_shape=jax.ShapeDtypeStruct((_B, _OUT), jnp.float32),
)


def kernel(x_cat_0, x_cat_1, x_cat_2, x_cat_3, x_cat_4, x_cat_5, x_cat_6,
           x_cat_7, x_cat_8, x_cat_9, x_cat_10, x_cat_11, x_cat_12,
           x_cat_13, x_cat_14, x_cat_15, x_cat_16, x_cat_17, x_cat_18,
           x_cat_19, x_cat_20, x_cat_21, x_cat_22, x_cat_23, x_cat_24,
           x_cat_25, x_mer_0, x_mer_1, x_num, cat_table_0, cat_table_1,
           cat_table_2, cat_table_3, cat_table_4, cat_table_5, cat_table_6,
           cat_table_7, cat_table_8, cat_table_9, cat_table_10,
           cat_table_11, cat_table_12, cat_table_13, cat_table_14,
           cat_table_15, cat_table_16, cat_table_17, cat_table_18,
           cat_table_19, cat_table_20, cat_table_21, cat_table_22,
           cat_table_23, cat_table_24, cat_table_25, mer_table_0,
           mer_table_1, W1, b1, W2, b2, W3, b3):
    kw = dict(locals())
    xcats = [kw["x_cat_%d" % i].astype(jnp.int32) for i in range(_N_CAT)]
    xmers = [
        kw["x_mer_%d" % j].astype(jnp.int32).reshape(_B * _HIST)
        for j in range(_N_MER)
    ]
    cat_tabs = [kw["cat_table_%d" % i] for i in range(_N_CAT)]
    mer_tabs = [kw["mer_table_%d" % j] for j in range(_N_MER)]

    feat = _sc_gather(*xcats, *xmers, *cat_tabs, *mer_tabs)
    w1a = W1[:_FEAT]
    w1b = W1[_FEAT:]
    return _mlp(feat, x_num, w1a, w1b, b1, W2, b2, W3, b3)


# speedup vs baseline: 1.2824x; 1.2824x over previous
import functools

import jax
import jax.numpy as jnp
import numpy as np
from jax import lax
from jax.experimental import pallas as pl
from jax.experimental.pallas import tpu as pltpu
from jax.experimental.pallas import tpu_sc as plsc

_N_CAT = 26
_CAT_DIM = 32
_N_MER = 2
_MER_DIM = 32
_B = 4096
_HIST = 50
_N_NUM = 13
_H1, _H2, _OUT = 512, 256, 2
_CFEAT = _N_CAT * _CAT_DIM
_CVOCAB = 100000

_NC, _NS = 2, 16
_NW = _NC * _NS
_BPW = _B // _NW
_CB = 8
_NCHUNK = _BPW // _CB
_PIECES = ((0, 128), (128, 128), (256, 128), (384, 16))

_mesh = plsc.VectorSubcoreMesh(
    core_axis_name="c", subcore_axis_name="s", num_cores=_NC, num_subcores=_NS
)


@functools.partial(
    pl.kernel,
    out_type=(
        jax.ShapeDtypeStruct((_CFEAT, _B), jnp.float32),
        jax.ShapeDtypeStruct((_B, _MER_DIM), jnp.float32),
        jax.ShapeDtypeStruct((_B, _MER_DIM), jnp.float32),
    ),
    mesh=_mesh,
    compiler_params=pltpu.CompilerParams(
        use_tc_tiling_on_sc=False, needs_layout_passes=False),
    scratch_types=[
        pltpu.VMEM((_B,), jnp.int32),
        pltpu.VMEM((_CVOCAB,), jnp.float32),
        pltpu.VMEM((_B,), jnp.float32),
        pltpu.VMEM((_CB * _HIST,), jnp.int32),
        pltpu.VMEM((_CB * _HIST, _MER_DIM), jnp.float32),
        pltpu.VMEM((_CB, _MER_DIM), jnp.float32),
        pltpu.SemaphoreType.DMA,
    ],
)
def _sc_gather(*refs):
    xcat = refs[0:_N_CAT]
    xmer = refs[_N_CAT:_N_CAT + _N_MER]
    catT = refs[_N_CAT + _N_MER:2 * _N_CAT + _N_MER]
    mer_tabs = refs[2 * _N_CAT + _N_MER:2 * _N_CAT + 2 * _N_MER]
    xT, m0, m1 = refs[2 * _N_CAT + 2 * _N_MER:2 * _N_CAT + 2 * _N_MER + 3]
    mouts = (m0, m1)
    (cidx_v, drow_v, xrow_v, midx_v, mrow_v, macc_v, sem) = refs[
        2 * _N_CAT + 2 * _N_MER + 3:]

    wid = lax.axis_index("s") * _NC + lax.axis_index("c")
    inv = np.float32(1.0 / _HIST)

    for t in range(_N_CAT):
        pltpu.sync_copy(xcat[t], cidx_v)
        pltpu.sync_copy(catT[t].at[wid, :], drow_v)

        def _gat(i, carry):
            o = i * 64
            for u in range(4):
                s = o + u * 16
                idx16 = cidx_v[pl.ds(s, 16)]
                xrow_v[pl.ds(s, 16)] = plsc.load_gather(drow_v, [idx16])
            return carry

        lax.fori_loop(0, _B // 64, _gat, 0)
        pltpu.sync_copy(xrow_v, xT.at[_CAT_DIM * t + wid, :])

    base = wid * _BPW
    for j in range(_N_MER):

        def _chunk(t, carry, j=j):
            start = base * _HIST + t * (_CB * _HIST)
            pltpu.sync_copy(xmer[j].at[pl.ds(start, _CB * _HIST)], midx_v)
            copies = [
                pltpu.async_copy(
                    mer_tabs[j].at[midx_v.at[pl.ds(off, ln)]],
                    mrow_v.at[pl.ds(off, ln)],
                    sem,
                )
                for off, ln in _PIECES
            ]
            for c in copies:
                c.wait()

            def _bag(b, carry2):
                r0 = b * _HIST
                a0 = mrow_v[r0, 0:16]
                a1 = mrow_v[r0, 16:32]
                b0 = mrow_v[r0 + 1, 0:16]
                b1 = mrow_v[r0 + 1, 16:32]
                for h in range(2, _HIST, 2):
                    a0 += mrow_v[r0 + h, 0:16]
                    a1 += mrow_v[r0 + h, 16:32]
                    b0 += mrow_v[r0 + h + 1, 0:16]
                    b1 += mrow_v[r0 + h + 1, 16:32]
                macc_v[b, 0:16] = (a0 + b0) * inv
                macc_v[b, 16:32] = (a1 + b1) * inv
                return carry2

            lax.fori_loop(0, _CB, _bag, 0)
            pltpu.sync_copy(macc_v, mouts[j].at[pl.ds(base + t * _CB, _CB), :])
            return carry

        lax.fori_loop(0, _NCHUNK, _chunk, 0)


_BT = 256


def _mlp_body(xt_ref, m0_ref, m1_ref, xn_ref, w1c_ref, w1m0_ref, w1m1_ref,
              w1n_ref, b1_ref, w2_ref, b2_ref, w3_ref, b3_ref, o_ref):
    h = lax.dot_general(
        xt_ref[...], w1c_ref[...], (((0,), (0,)), ((), ())),
        preferred_element_type=jnp.float32,
    )
    h += jnp.dot(m0_ref[...], w1m0_ref[...], preferred_element_type=jnp.float32)
    h += jnp.dot(m1_ref[...], w1m1_ref[...], preferred_element_type=jnp.float32)
    h += jnp.dot(xn_ref[...], w1n_ref[...], preferred_element_type=jnp.float32)
    h = jnp.maximum(h + b1_ref[...], 0.0)
    h = jnp.maximum(
        jnp.dot(h, w2_ref[...], preferred_element_type=jnp.float32) + b2_ref[...],
        0.0,
    )
    o_ref[...] = (
        jnp.dot(h, w3_ref[...], preferred_element_type=jnp.float32) + b3_ref[...]
    )


_mlp = pl.pallas_call(
    _mlp_body,
    grid=(_B // _BT,),
    in_specs=[
        pl.BlockSpec((_CFEAT, _BT), lambda i: (0, i)),
        pl.BlockSpec((_BT, _MER_DIM), lambda i: (i, 0)),
        pl.BlockSpec((_BT, _MER_DIM), lambda i: (i, 0)),
        pl.BlockSpec((_BT, _N_NUM), lambda i: (i, 0)),
        pl.BlockSpec((_CFEAT, _H1), lambda i: (0, 0)),
        pl.BlockSpec((_MER_DIM, _H1), lambda i: (0, 0)),
        pl.BlockSpec((_MER_DIM, _H1), lambda i: (0, 0)),
        pl.BlockSpec((_N_NUM, _H1), lambda i: (0, 0)),
        pl.BlockSpec((_H1,), lambda i: (0,)),
        pl.BlockSpec((_H1, _H2), lambda i: (0, 0)),
        pl.BlockSpec((_H2,), lambda i: (0,)),
        pl.BlockSpec((_H2, _OUT), lambda i: (0, 0)),
        pl.BlockSpec((_OUT,), lambda i: (0,)),
    ],
    out_specs=pl.BlockSpec((_BT, _OUT), lambda i: (i, 0)),
    out_shape=jax.ShapeDtypeStruct((_B, _OUT), jnp.float32),
)


def kernel(x_cat_0, x_cat_1, x_cat_2, x_cat_3, x_cat_4, x_cat_5, x_cat_6,
           x_cat_7, x_cat_8, x_cat_9, x_cat_10, x_cat_11, x_cat_12,
           x_cat_13, x_cat_14, x_cat_15, x_cat_16, x_cat_17, x_cat_18,
           x_cat_19, x_cat_20, x_cat_21, x_cat_22, x_cat_23, x_cat_24,
           x_cat_25, x_mer_0, x_mer_1, x_num, cat_table_0, cat_table_1,
           cat_table_2, cat_table_3, cat_table_4, cat_table_5, cat_table_6,
           cat_table_7, cat_table_8, cat_table_9, cat_table_10,
           cat_table_11, cat_table_12, cat_table_13, cat_table_14,
           cat_table_15, cat_table_16, cat_table_17, cat_table_18,
           cat_table_19, cat_table_20, cat_table_21, cat_table_22,
           cat_table_23, cat_table_24, cat_table_25, mer_table_0,
           mer_table_1, W1, b1, W2, b2, W3, b3):
    kw = dict(locals())
    xcats = [kw["x_cat_%d" % i].astype(jnp.int32) for i in range(_N_CAT)]
    xmers = [
        kw["x_mer_%d" % j].astype(jnp.int32).reshape(_B * _HIST)
        for j in range(_N_MER)
    ]
    catTs = [kw["cat_table_%d" % i].T for i in range(_N_CAT)]
    mer_tabs = [kw["mer_table_%d" % j] for j in range(_N_MER)]

    xT, m0, m1 = _sc_gather(*xcats, *xmers, *catTs, *mer_tabs)
    w1c = W1[:_CFEAT]
    w1m0 = W1[_CFEAT:_CFEAT + _MER_DIM]
    w1m1 = W1[_CFEAT + _MER_DIM:_CFEAT + 2 * _MER_DIM]
    w1n = W1[_CFEAT + 2 * _MER_DIM:]
    return _mlp(xT, m0, m1, x_num, w1c, w1m0, w1m1, w1n, b1, W2, b2, W3, b3)


# speedup vs baseline: 1.4386x; 1.1219x over previous
import functools

import jax
import jax.numpy as jnp
import numpy as np
from jax import lax
from jax.experimental import pallas as pl
from jax.experimental.pallas import tpu as pltpu
from jax.experimental.pallas import tpu_sc as plsc

_N_CAT = 26
_CAT_DIM = 32
_N_MER = 2
_MER_DIM = 32
_B = 4096
_HIST = 50
_N_NUM = 13
_H1, _H2, _OUT = 512, 256, 2
_CFEAT = _N_CAT * _CAT_DIM
_CVOCAB = 100000

_NC, _NS = 2, 16
_NW = _NC * _NS
_BPW = _B // _NW
_HH = _HIST // 2

_mesh = plsc.VectorSubcoreMesh(
    core_axis_name="c", subcore_axis_name="s", num_cores=_NC, num_subcores=_NS
)
_sc_params = pltpu.CompilerParams(
    use_tc_tiling_on_sc=False, needs_layout_passes=False)


@functools.partial(
    pl.kernel,
    out_type=(
        jax.ShapeDtypeStruct((_B, _MER_DIM), jnp.float32),
        jax.ShapeDtypeStruct((_B, _MER_DIM), jnp.float32),
    ),
    mesh=_mesh,
    compiler_params=_sc_params,
    scratch_types=[
        pltpu.VMEM((_HIST, _BPW), jnp.int32),
        pltpu.VMEM((_HH * _BPW, _MER_DIM), jnp.float32),
        pltpu.VMEM((_BPW, _MER_DIM), jnp.float32),
        pltpu.SemaphoreType.DMA,
    ],
)
def _sc_mer(*refs):
    xmer = refs[0:_N_MER]
    mer_tabs = refs[_N_MER:2 * _N_MER]
    mouts = refs[2 * _N_MER:2 * _N_MER + 2]
    midx_v, mrow_v, macc_v, sem = refs[2 * _N_MER + 2:]

    wid = lax.axis_index("s") * _NC + lax.axis_index("c")
    base = wid * _BPW
    inv = np.float32(1.0 / _HIST)

    for j in range(_N_MER):
        loads = [
            pltpu.async_copy(
                xmer[j].at[pl.ds(h * _B + base, _BPW)], midx_v.at[h], sem)
            for h in range(_HIST)
        ]
        for c in loads:
            c.wait()

        for t in range(2):
            copies = [
                pltpu.async_copy(
                    mer_tabs[j].at[midx_v.at[t * _HH + h]],
                    mrow_v.at[pl.ds(h * _BPW, _BPW)],
                    sem,
                )
                for h in range(_HH)
            ]
            for c in copies:
                c.wait()

            def _bag(b, carry, t=t, j=j):
                a0 = mrow_v[b, 0:16]
                a1 = mrow_v[b, 16:32]
                b0 = mrow_v[_BPW + b, 0:16]
                b1 = mrow_v[_BPW + b, 16:32]
                for h in range(2, _HH - 1, 2):
                    r = h * _BPW + b
                    a0 += mrow_v[r, 0:16]
                    a1 += mrow_v[r, 16:32]
                    b0 += mrow_v[r + _BPW, 0:16]
                    b1 += mrow_v[r + _BPW, 16:32]
                r_last = (_HH - 1) * _BPW + b
                s0 = a0 + b0 + mrow_v[r_last, 0:16]
                s1 = a1 + b1 + mrow_v[r_last, 16:32]
                if t == 0:
                    macc_v[b, 0:16] = s0
                    macc_v[b, 16:32] = s1
                else:
                    macc_v[b, 0:16] = (macc_v[b, 0:16] + s0) * inv
                    macc_v[b, 16:32] = (macc_v[b, 16:32] + s1) * inv
                return carry

            lax.fori_loop(0, _BPW, _bag, 0)
        pltpu.sync_copy(macc_v, mouts[j].at[pl.ds(base, _BPW), :])


@functools.partial(
    pl.kernel,
    out_type=jax.ShapeDtypeStruct((_CFEAT, _B), jnp.float32),
    mesh=_mesh,
    compiler_params=_sc_params,
    scratch_types=[
        pltpu.VMEM((_B,), jnp.int32),
        pltpu.VMEM((_CVOCAB,), jnp.float32),
        pltpu.VMEM((_B,), jnp.float32),
        pltpu.SemaphoreType.DMA,
    ],
)
def _sc_cat(*refs):
    xcat = refs[0:_N_CAT]
    catT = refs[_N_CAT:2 * _N_CAT]
    xT = refs[2 * _N_CAT]
    cidx_v, drow_v, xrow_v, sem = refs[2 * _N_CAT + 1:]

    wid = lax.axis_index("s") * _NC + lax.axis_index("c")

    for t in range(_N_CAT):
        pltpu.sync_copy(xcat[t], cidx_v)
        pltpu.sync_copy(catT[t].at[wid, :], drow_v)

        def _gat(i, carry):
            o = i * 64
            for u in range(4):
                s = o + u * 16
                idx16 = cidx_v[pl.ds(s, 16)]
                xrow_v[pl.ds(s, 16)] = plsc.load_gather(drow_v, [idx16])
            return carry

        lax.fori_loop(0, _B // 64, _gat, 0)
        pltpu.sync_copy(xrow_v, xT.at[_CAT_DIM * t + wid, :])


_BT = 256


def _mlp_body(xt_ref, m0_ref, m1_ref, xn_ref, w1c_ref, w1m0_ref, w1m1_ref,
              w1n_ref, b1_ref, w2_ref, b2_ref, w3_ref, b3_ref, o_ref):
    h = lax.dot_general(
        xt_ref[...], w1c_ref[...], (((0,), (0,)), ((), ())),
        preferred_element_type=jnp.float32,
    )
    h += jnp.dot(m0_ref[...], w1m0_ref[...], preferred_element_type=jnp.float32)
    h += jnp.dot(m1_ref[...], w1m1_ref[...], preferred_element_type=jnp.float32)
    h += jnp.dot(xn_ref[...], w1n_ref[...], preferred_element_type=jnp.float32)
    h = jnp.maximum(h + b1_ref[...], 0.0)
    h = jnp.maximum(
        jnp.dot(h, w2_ref[...], preferred_element_type=jnp.float32) + b2_ref[...],
        0.0,
    )
    o_ref[...] = (
        jnp.dot(h, w3_ref[...], preferred_element_type=jnp.float32) + b3_ref[...]
    )


_mlp = pl.pallas_call(
    _mlp_body,
    grid=(_B // _BT,),
    in_specs=[
        pl.BlockSpec((_CFEAT, _BT), lambda i: (0, i)),
        pl.BlockSpec((_BT, _MER_DIM), lambda i: (i, 0)),
        pl.BlockSpec((_BT, _MER_DIM), lambda i: (i, 0)),
        pl.BlockSpec((_BT, _N_NUM), lambda i: (i, 0)),
        pl.BlockSpec((_CFEAT, _H1), lambda i: (0, 0)),
        pl.BlockSpec((_MER_DIM, _H1), lambda i: (0, 0)),
        pl.BlockSpec((_MER_DIM, _H1), lambda i: (0, 0)),
        pl.BlockSpec((_N_NUM, _H1), lambda i: (0, 0)),
        pl.BlockSpec((_H1,), lambda i: (0,)),
        pl.BlockSpec((_H1, _H2), lambda i: (0, 0)),
        pl.BlockSpec((_H2,), lambda i: (0,)),
        pl.BlockSpec((_H2, _OUT), lambda i: (0, 0)),
        pl.BlockSpec((_OUT,), lambda i: (0,)),
    ],
    out_specs=pl.BlockSpec((_BT, _OUT), lambda i: (i, 0)),
    out_shape=jax.ShapeDtypeStruct((_B, _OUT), jnp.float32),
)


def kernel(x_cat_0, x_cat_1, x_cat_2, x_cat_3, x_cat_4, x_cat_5, x_cat_6,
           x_cat_7, x_cat_8, x_cat_9, x_cat_10, x_cat_11, x_cat_12,
           x_cat_13, x_cat_14, x_cat_15, x_cat_16, x_cat_17, x_cat_18,
           x_cat_19, x_cat_20, x_cat_21, x_cat_22, x_cat_23, x_cat_24,
           x_cat_25, x_mer_0, x_mer_1, x_num, cat_table_0, cat_table_1,
           cat_table_2, cat_table_3, cat_table_4, cat_table_5, cat_table_6,
           cat_table_7, cat_table_8, cat_table_9, cat_table_10,
           cat_table_11, cat_table_12, cat_table_13, cat_table_14,
           cat_table_15, cat_table_16, cat_table_17, cat_table_18,
           cat_table_19, cat_table_20, cat_table_21, cat_table_22,
           cat_table_23, cat_table_24, cat_table_25, mer_table_0,
           mer_table_1, W1, b1, W2, b2, W3, b3):
    kw = dict(locals())
    xcats = [kw["x_cat_%d" % i].astype(jnp.int32) for i in range(_N_CAT)]
    xmers = [
        kw["x_mer_%d" % j].astype(jnp.int32).T.reshape(_B * _HIST)
        for j in range(_N_MER)
    ]
    catTs = [kw["cat_table_%d" % i].T for i in range(_N_CAT)]
    mer_tabs = [kw["mer_table_%d" % j] for j in range(_N_MER)]

    m0, m1 = _sc_mer(*xmers, *mer_tabs)
    xT = _sc_cat(*xcats, *catTs)
    w1c = W1[:_CFEAT]
    w1m0 = W1[_CFEAT:_CFEAT + _MER_DIM]
    w1m1 = W1[_CFEAT + _MER_DIM:_CFEAT + 2 * _MER_DIM]
    w1n = W1[_CFEAT + 2 * _MER_DIM:]
    return _mlp(xT, m0, m1, x_num, w1c, w1m0, w1m1, w1n, b1, W2, b2, W3, b3)


# speedup vs baseline: 1.4408x; 1.0015x over previous
import functools

import jax
import jax.numpy as jnp
import numpy as np
from jax import lax
from jax.experimental import pallas as pl
from jax.experimental.pallas import tpu as pltpu
from jax.experimental.pallas import tpu_sc as plsc

_N_CAT = 26
_CAT_DIM = 32
_N_MER = 2
_MER_DIM = 32
_B = 4096
_HIST = 50
_N_NUM = 13
_H1, _H2, _OUT = 512, 256, 2
_CFEAT = _N_CAT * _CAT_DIM
_CVOCAB = 100000

_NC, _NS = 2, 16
_NW = _NC * _NS
_BPW = _B // _NW
_HH = _HIST // 2

_mesh = plsc.VectorSubcoreMesh(
    core_axis_name="c", subcore_axis_name="s", num_cores=_NC, num_subcores=_NS
)
_sc_params = pltpu.CompilerParams(
    use_tc_tiling_on_sc=False, needs_layout_passes=False)


@functools.partial(
    pl.kernel,
    out_type=(
        jax.ShapeDtypeStruct((_B, _MER_DIM), jnp.float32),
        jax.ShapeDtypeStruct((_B, _MER_DIM), jnp.float32),
    ),
    mesh=_mesh,
    compiler_params=_sc_params,
    scratch_types=[
        pltpu.VMEM((_HIST, _BPW), jnp.int32),
        pltpu.VMEM((_HH * _BPW, _MER_DIM), jnp.float32),
        pltpu.VMEM((_BPW, _MER_DIM), jnp.float32),
        pltpu.SemaphoreType.DMA,
    ],
)
def _sc_mer(*refs):
    xmer = refs[0:_N_MER]
    mer_tabs = refs[_N_MER:2 * _N_MER]
    mouts = refs[2 * _N_MER:2 * _N_MER + 2]
    midx_v, mrow_v, macc_v, sem = refs[2 * _N_MER + 2:]

    wid = lax.axis_index("s") * _NC + lax.axis_index("c")
    base = wid * _BPW
    inv = np.float32(1.0 / _HIST)

    for j in range(_N_MER):
        pltpu.sync_copy(xmer[j].at[:, pl.ds(base, _BPW)], midx_v)

        for t in range(2):
            copies = [
                pltpu.async_copy(
                    mer_tabs[j].at[midx_v.at[t * _HH + h]],
                    mrow_v.at[pl.ds(h * _BPW, _BPW)],
                    sem,
                )
                for h in range(_HH)
            ]
            for c in copies:
                c.wait()

            def _bag(b, carry, t=t, j=j):
                a0 = mrow_v[b, 0:16]
                a1 = mrow_v[b, 16:32]
                b0 = mrow_v[_BPW + b, 0:16]
                b1 = mrow_v[_BPW + b, 16:32]
                for h in range(2, _HH - 1, 2):
                    r = h * _BPW + b
                    a0 += mrow_v[r, 0:16]
                    a1 += mrow_v[r, 16:32]
                    b0 += mrow_v[r + _BPW, 0:16]
                    b1 += mrow_v[r + _BPW, 16:32]
                r_last = (_HH - 1) * _BPW + b
                s0 = a0 + b0 + mrow_v[r_last, 0:16]
                s1 = a1 + b1 + mrow_v[r_last, 16:32]
                if t == 0:
                    macc_v[b, 0:16] = s0
                    macc_v[b, 16:32] = s1
                else:
                    macc_v[b, 0:16] = (macc_v[b, 0:16] + s0) * inv
                    macc_v[b, 16:32] = (macc_v[b, 16:32] + s1) * inv
                return carry

            lax.fori_loop(0, _BPW, _bag, 0)
        pltpu.sync_copy(macc_v, mouts[j].at[pl.ds(base, _BPW), :])


@functools.partial(
    pl.kernel,
    out_type=jax.ShapeDtypeStruct((_CFEAT, _B), jnp.float32),
    mesh=_mesh,
    compiler_params=_sc_params,
    scratch_types=[
        pltpu.VMEM((_B,), jnp.int32),
        pltpu.VMEM((_CVOCAB,), jnp.float32),
        pltpu.VMEM((_B,), jnp.float32),
        pltpu.SemaphoreType.DMA,
    ],
)
def _sc_cat(*refs):
    xcat = refs[0:_N_CAT]
    catT = refs[_N_CAT:2 * _N_CAT]
    xT = refs[2 * _N_CAT]
    cidx_v, drow_v, xrow_v, sem = refs[2 * _N_CAT + 1:]

    wid = lax.axis_index("s") * _NC + lax.axis_index("c")

    for t in range(_N_CAT):
        pltpu.sync_copy(xcat[t], cidx_v)
        pltpu.sync_copy(catT[t].at[wid, :], drow_v)

        def _gat(i, carry):
            o = i * 64
            for u in range(4):
                s = o + u * 16
                idx16 = cidx_v[pl.ds(s, 16)]
                xrow_v[pl.ds(s, 16)] = plsc.load_gather(drow_v, [idx16])
            return carry

        lax.fori_loop(0, _B // 64, _gat, 0)
        pltpu.sync_copy(xrow_v, xT.at[_CAT_DIM * t + wid, :])


_BT = 256


def _mlp_body(xt_ref, m0_ref, m1_ref, xn_ref, w1c_ref, w1m0_ref, w1m1_ref,
              w1n_ref, b1_ref, w2_ref, b2_ref, w3_ref, b3_ref, o_ref):
    h = lax.dot_general(
        xt_ref[...], w1c_ref[...], (((0,), (0,)), ((), ())),
        preferred_element_type=jnp.float32,
    )
    h += jnp.dot(m0_ref[...], w1m0_ref[...], preferred_element_type=jnp.float32)
    h += jnp.dot(m1_ref[...], w1m1_ref[...], preferred_element_type=jnp.float32)
    h += jnp.dot(xn_ref[...], w1n_ref[...], preferred_element_type=jnp.float32)
    h = jnp.maximum(h + b1_ref[...], 0.0)
    h = jnp.maximum(
        jnp.dot(h, w2_ref[...], preferred_element_type=jnp.float32) + b2_ref[...],
        0.0,
    )
    o_ref[...] = (
        jnp.dot(h, w3_ref[...], preferred_element_type=jnp.float32) + b3_ref[...]
    )


_mlp = pl.pallas_call(
    _mlp_body,
    grid=(_B // _BT,),
    in_specs=[
        pl.BlockSpec((_CFEAT, _BT), lambda i: (0, i)),
        pl.BlockSpec((_BT, _MER_DIM), lambda i: (i, 0)),
        pl.BlockSpec((_BT, _MER_DIM), lambda i: (i, 0)),
        pl.BlockSpec((_BT, _N_NUM), lambda i: (i, 0)),
        pl.BlockSpec((_CFEAT, _H1), lambda i: (0, 0)),
        pl.BlockSpec((_MER_DIM, _H1), lambda i: (0, 0)),
        pl.BlockSpec((_MER_DIM, _H1), lambda i: (0, 0)),
        pl.BlockSpec((_N_NUM, _H1), lambda i: (0, 0)),
        pl.BlockSpec((_H1,), lambda i: (0,)),
        pl.BlockSpec((_H1, _H2), lambda i: (0, 0)),
        pl.BlockSpec((_H2,), lambda i: (0,)),
        pl.BlockSpec((_H2, _OUT), lambda i: (0, 0)),
        pl.BlockSpec((_OUT,), lambda i: (0,)),
    ],
    out_specs=pl.BlockSpec((_BT, _OUT), lambda i: (i, 0)),
    out_shape=jax.ShapeDtypeStruct((_B, _OUT), jnp.float32),
)


def kernel(x_cat_0, x_cat_1, x_cat_2, x_cat_3, x_cat_4, x_cat_5, x_cat_6,
           x_cat_7, x_cat_8, x_cat_9, x_cat_10, x_cat_11, x_cat_12,
           x_cat_13, x_cat_14, x_cat_15, x_cat_16, x_cat_17, x_cat_18,
           x_cat_19, x_cat_20, x_cat_21, x_cat_22, x_cat_23, x_cat_24,
           x_cat_25, x_mer_0, x_mer_1, x_num, cat_table_0, cat_table_1,
           cat_table_2, cat_table_3, cat_table_4, cat_table_5, cat_table_6,
           cat_table_7, cat_table_8, cat_table_9, cat_table_10,
           cat_table_11, cat_table_12, cat_table_13, cat_table_14,
           cat_table_15, cat_table_16, cat_table_17, cat_table_18,
           cat_table_19, cat_table_20, cat_table_21, cat_table_22,
           cat_table_23, cat_table_24, cat_table_25, mer_table_0,
           mer_table_1, W1, b1, W2, b2, W3, b3):
    kw = dict(locals())
    xcats = [kw["x_cat_%d" % i].astype(jnp.int32) for i in range(_N_CAT)]
    xmers = [kw["x_mer_%d" % j].astype(jnp.int32).T for j in range(_N_MER)]
    catTs = [kw["cat_table_%d" % i].T for i in range(_N_CAT)]
    mer_tabs = [kw["mer_table_%d" % j] for j in range(_N_MER)]

    m0, m1 = _sc_mer(*xmers, *mer_tabs)
    xT = _sc_cat(*xcats, *catTs)
    w1c = W1[:_CFEAT]
    w1m0 = W1[_CFEAT:_CFEAT + _MER_DIM]
    w1m1 = W1[_CFEAT + _MER_DIM:_CFEAT + 2 * _MER_DIM]
    w1n = W1[_CFEAT + 2 * _MER_DIM:]
    return _mlp(xT, m0, m1, x_num, w1c, w1m0, w1m1, w1n, b1, W2, b2, W3, b3)
